# baseline (speedup 1.0000x reference)
import functools

import jax
import jax.numpy as jnp
from jax import lax
from jax.experimental import pallas as pl
from jax.experimental.pallas import tpu as pltpu
from jax.experimental.pallas import tpu_sc as plsc

_NC = 2
_NS = 16
_NW = _NC * _NS
_IW = 128


def _gelu(x):
    return 0.5 * x * (1.0 + lax.erf(x * 0.7071067811865476))


def _ln(x, g, b):
    m = jnp.mean(x, axis=-1, keepdims=True)
    v = jnp.mean((x - m) ** 2, axis=-1, keepdims=True)
    return (x - m) / jnp.sqrt(v + 1e-5) * g + b


def _sc_gather_rows(table, idx3):
    _, dt = table.shape
    nq, gsub, iw = idx3.shape
    m = nq * gsub * iw
    ge = gsub * iw
    mesh = plsc.VectorSubcoreMesh(core_axis_name="c", subcore_axis_name="s")

    @functools.partial(
        pl.kernel, mesh=mesh,
        out_type=jax.ShapeDtypeStruct((m, dt), table.dtype),
        scratch_types=[
            pltpu.VMEM((gsub, iw), jnp.int32),
            pltpu.VMEM((ge, dt), table.dtype),
            pltpu.SemaphoreType.DMA,
        ],
    )
    def k(table_hbm, idx_hbm, out_hbm, idx_v, rows_v, sem):
        wid = lax.axis_index("s") * _NC + lax.axis_index("c")

        @pl.loop(wid, nq, step=_NW)
        def _(q):
            pltpu.sync_copy(idx_hbm.at[q], idx_v)
            waits = [
                pltpu.async_copy(table_hbm.at[idx_v.at[j]],
                                 rows_v.at[pl.ds(j * iw, iw)], sem)
                for j in range(gsub)
            ]
            for w in waits:
                w.wait()
            pltpu.sync_copy(rows_v, out_hbm.at[pl.ds(q * ge, ge)])

    return k(table, idx3)


_GSUB = 4
_GE = _GSUB * _IW
_ZC = 40


def _sc_scatter_add(vals, idx3, nseg):
    m, dv = vals.shape
    assert dv == 128
    ngrp = m // _GE
    half = nseg // _NC
    acc_rows = half + _ZC
    nzc = acc_rows // _ZC
    nxc = half // _ZC
    assert m % _GE == 0 and half % _ZC == 0
    mesh = plsc.VectorSubcoreMesh(core_axis_name="c", subcore_axis_name="s")

    @functools.partial(
        pl.kernel, mesh=mesh,
        out_type=jax.ShapeDtypeStruct((nseg, dv), jnp.float32),
        scratch_types=[
            pltpu.VMEM((_GSUB, _IW), jnp.int32),
            pltpu.VMEM((_GE, dv), jnp.float32),
            pltpu.VMEM((_ZC, dv), jnp.float32),
            pltpu.VMEM_SHARED((acc_rows, dv), jnp.float32),
            pltpu.SemaphoreType.DMA,
        ],
    )
    def k(vals_hbm, idx_hbm, out_hbm, idx_v, data_v, zero_v, acc_sh, sem):
        cid = lax.axis_index("c")
        sid = lax.axis_index("s")
        lo = cid * half

        @pl.loop(0, _ZC)
        def _(r):
            @pl.loop(0, dv, step=16)
            def _(c0):
                zero_v[r, pl.ds(c0, 16)] = jnp.zeros((16,), jnp.float32)

        @pl.loop(sid, nzc, step=_NS)
        def _(c0):
            pltpu.sync_copy(zero_v, acc_sh.at[pl.ds(c0 * _ZC, _ZC)])

        plsc.subcore_barrier()

        @pl.loop(sid, ngrp, step=_NS)
        def _(g):
            pltpu.sync_copy(idx_hbm.at[g], idx_v)
            pltpu.sync_copy(vals_hbm.at[pl.ds(g * _GE, _GE)], data_v)
            @pl.loop(0, _GSUB)
            def _(j):
                @pl.loop(0, _IW, step=16)
                def _(k0):
                    v = idx_v[j, pl.ds(k0, 16)] - lo
                    bad = (v < 0) | (v >= half)
                    idx_v[j, pl.ds(k0, 16)] = jnp.where(bad, half, v)

            for j in range(_GSUB):
                pltpu.sync_copy(data_v.at[pl.ds(j * _IW, _IW)],
                                acc_sh.at[idx_v.at[j]], add=True)

        plsc.subcore_barrier()

        @pl.loop(sid, nxc, step=_NS)
        def _(c0):
            pltpu.sync_copy(
                acc_sh.at[pl.ds(c0 * _ZC, _ZC)],
                out_hbm.at[pl.ds(lo + c0 * _ZC, _ZC)],
            )

    return k(vals, idx3)


def _edge_compute(gath2, eattr, wcat, bcat, abd, ab_row, n2_w, nb2, n3_w, nb3,
                  e_num, be_blk):
    nblk = e_num // be_blk
    d = 128
    hd = 512

    bf = jnp.bfloat16

    def body(gs_r, gk_r, ea_r, w_r, b_r, a_r, ab_r, n2_r, nb2_r, n3_r, nb3_r,
             ew_r, nu_r):
        pre = (
            jnp.dot(gs_r[...].astype(bf), w_r[0:d, :], preferred_element_type=jnp.float32)
            + jnp.dot(gk_r[...].astype(bf), w_r[d:2 * d, :], preferred_element_type=jnp.float32)
            + jnp.dot(ea_r[...].astype(bf), w_r[2 * d:, :], preferred_element_type=jnp.float32)
            + b_r[...]
        )
        att = pre[:, 0:hd]
        act = jnp.where(att >= 0, att, 0.2 * att)
        aw = jnp.dot(act.astype(bf), a_r[...], preferred_element_type=jnp.float32) + ab_r[...]
        lane = lax.broadcasted_iota(jnp.int32, (be_blk, 128), 1)
        ew_r[...] = jnp.where(lane < 4, jnp.exp(aw) + 1e-12, 0.0)
        h = _gelu(pre[:, hd:hd + d])
        h = _gelu(jnp.dot(h.astype(bf), n2_r[...], preferred_element_type=jnp.float32) + nb2_r[...])
        nu_r[...] = (jnp.dot(h.astype(bf), n3_r[...], preferred_element_type=jnp.float32) + nb3_r[...]).astype(bf)

    return pl.pallas_call(
        body,
        grid=(nblk,),
        in_specs=[
            pl.BlockSpec((be_blk, d), lambda i: (i, 0)),
            pl.BlockSpec((be_blk, d), lambda i, n=nblk: (i + n, 0)),
            pl.BlockSpec((be_blk, 16), lambda i: (i, 0)),
            pl.BlockSpec(wcat.shape, lambda i: (0, 0)),
            pl.BlockSpec(bcat.shape, lambda i: (0, 0)),
            pl.BlockSpec(abd.shape, lambda i: (0, 0)),
            pl.BlockSpec(ab_row.shape, lambda i: (0, 0)),
            pl.BlockSpec(n2_w.shape, lambda i: (0, 0)),
            pl.BlockSpec(nb2.shape, lambda i: (0, 0)),
            pl.BlockSpec(n3_w.shape, lambda i: (0, 0)),
            pl.BlockSpec(nb3.shape, lambda i: (0, 0)),
        ],
        out_specs=[
            pl.BlockSpec((be_blk, 128), lambda i: (i, 0)),
            pl.BlockSpec((be_blk, d), lambda i: (i, 0)),
        ],
        out_shape=[
            jax.ShapeDtypeStruct((e_num, 128), jnp.float32),
            jax.ShapeDtypeStruct((e_num, 128), jnp.bfloat16),
        ],
    )(gath2, gath2, eattr, wcat, bcat, abd, ab_row, n2_w, nb2, n3_w, nb3)


def _combine_norm(normtab, n_num, bn_blk):
    nblk = n_num // bn_blk

    def body(a_r, o_r):
        s = a_r[...]
        o_r[...] = jnp.where(s > 0, 0.25 / s, 0.0)

    return pl.pallas_call(
        body,
        grid=(nblk,),
        in_specs=[pl.BlockSpec((bn_blk, 128), lambda i: (i, 0))],
        out_specs=pl.BlockSpec((bn_blk, 128), lambda i: (i, 0)),
        out_shape=jax.ShapeDtypeStruct((n_num, 128), jnp.float32),
    )(normtab)


def _apply_atten(ew, rn, nu, e_num, be_blk):
    nblk = e_num // be_blk

    def body(ew_r, rn_r, nu_r, o_r):
        w = jnp.sum(ew_r[...] * rn_r[...], axis=1, keepdims=True)
        o_r[...] = w * nu_r[...].astype(jnp.float32)

    return pl.pallas_call(
        body,
        grid=(nblk,),
        in_specs=[
            pl.BlockSpec((be_blk, 128), lambda i: (i, 0)),
            pl.BlockSpec((be_blk, 128), lambda i: (i, 0)),
            pl.BlockSpec((be_blk, 128), lambda i: (i, 0)),
        ],
        out_specs=pl.BlockSpec((be_blk, 128), lambda i: (i, 0)),
        out_shape=jax.ShapeDtypeStruct((e_num, 128), jnp.float32),
    )(ew, rn, nu)


def _node_update(aggflat, bb, d1_w, d1_b, d2_w, d2_b, g1, b1, n_num, bn_blk):
    nblk = n_num // bn_blk

    bf = jnp.bfloat16

    def body(a0_r, bb_r, d1_r, d1b_r, d2_r, d2b_r, g_r, bb1_r, o_r):
        upd = a0_r[...]
        x = _ln(bb_r[...] + upd, g_r[...], bb1_r[...])
        h = _gelu(jnp.dot(x.astype(bf), d1_r[...], preferred_element_type=jnp.float32) + d1b_r[...])
        dense = jnp.dot(h.astype(bf), d2_r[...], preferred_element_type=jnp.float32) + d2b_r[...]
        o_r[...] = _ln(dense + upd, g_r[...], bb1_r[...])

    return pl.pallas_call(
        body,
        grid=(nblk,),
        in_specs=[
            pl.BlockSpec((bn_blk, 128), lambda i: (i, 0)),
            pl.BlockSpec((bn_blk, 128), lambda i: (i, 0)),
            pl.BlockSpec(d1_w.shape, lambda i: (0, 0)),
            pl.BlockSpec(d1_b.shape, lambda i: (0, 0)),
            pl.BlockSpec(d2_w.shape, lambda i: (0, 0)),
            pl.BlockSpec(d2_b.shape, lambda i: (0, 0)),
            pl.BlockSpec(g1.shape, lambda i: (0, 0)),
            pl.BlockSpec(b1.shape, lambda i: (0, 0)),
        ],
        out_specs=pl.BlockSpec((bn_blk, 128), lambda i: (i, 0)),
        out_shape=jax.ShapeDtypeStruct((n_num, 128), jnp.float32),
    )(aggflat, bb, d1_w, d1_b, d2_w, d2_b, g1, b1)


def _edge_update(gath2, eattr, e1_w, e1_b, e2_w, e2_b, e3_w, e3_b, ge, be, e_num, be_blk):
    nblk = e_num // be_blk
    d = 128

    bf = jnp.bfloat16

    def body(gs_r, gk_r, ea_r, e1_r, e1b_r, e2_r, e2b_r, e3_r, e3b_r, g_r, b_r, o_r):
        t = (
            jnp.dot(gs_r[...].astype(bf), e1_r[0:d, :], preferred_element_type=jnp.float32)
            + jnp.dot(gk_r[...].astype(bf), e1_r[d:2 * d, :], preferred_element_type=jnp.float32)
            + jnp.dot(ea_r[...].astype(bf), e1_r[2 * d:, :], preferred_element_type=jnp.float32)
            + e1b_r[...]
        )
        t = _gelu(t)
        t = _gelu(jnp.dot(t.astype(bf), e2_r[...], preferred_element_type=jnp.float32) + e2b_r[...])
        t = jnp.dot(t.astype(bf), e3_r[...], preferred_element_type=jnp.float32) + e3b_r[...]
        o_r[...] = _ln(ea_r[...] + t, g_r[...], b_r[...])

    return pl.pallas_call(
        body,
        grid=(nblk,),
        in_specs=[
            pl.BlockSpec((be_blk, d), lambda i: (i, 0)),
            pl.BlockSpec((be_blk, d), lambda i, n=nblk: (i + n, 0)),
            pl.BlockSpec((be_blk, 16), lambda i: (i, 0)),
            pl.BlockSpec(e1_w.shape, lambda i: (0, 0)),
            pl.BlockSpec(e1_b.shape, lambda i: (0, 0)),
            pl.BlockSpec(e2_w.shape, lambda i: (0, 0)),
            pl.BlockSpec(e2_b.shape, lambda i: (0, 0)),
            pl.BlockSpec(e3_w.shape, lambda i: (0, 0)),
            pl.BlockSpec(e3_b.shape, lambda i: (0, 0)),
            pl.BlockSpec(ge.shape, lambda i: (0, 0)),
            pl.BlockSpec(be.shape, lambda i: (0, 0)),
        ],
        out_specs=pl.BlockSpec((be_blk, 16), lambda i: (i, 0)),
        out_shape=jax.ShapeDtypeStruct((e_num, 16), jnp.float32),
    )(gath2, gath2, eattr, e1_w, e1_b, e2_w, e2_b, e3_w, e3_b, ge, be)


def kernel(bb_nodes, eidx, eattr, aW_w, aW_b, aA_w, aA_b, n1_w, n1_b, n2_w, n2_b,
           n3_w, n3_b, d1_w, d1_b, d2_w, d2_b, e1_w, e1_b, e2_w, e2_b, e3_w, e3_b,
           g1, b1, ge, be):
    n_num, d = bb_nodes.shape
    e_num, de = eattr.shape
    h = aW_w.shape[0]
    hd = h * d
    be_blk = 5000
    bn_blk = 2000

    snk = eidx[1]
    snk3 = snk.reshape(e_num // _GE, _GSUB, _IW)
    idx_both3 = eidx.reshape(2 * e_num // _GE, _GSUB, _IW)

    wcat = jnp.concatenate(
        [jnp.transpose(aW_w, (1, 0, 2)).reshape(2 * d + de, hd), n1_w], axis=1)
    bcat = jnp.concatenate(
        [aW_b.reshape(1, hd), n1_b.reshape(1, d)], axis=1)
    abd = jnp.zeros((hd, 128), jnp.float32).at[
        jnp.arange(hd), jnp.repeat(jnp.arange(h), d)].set(aA_w.reshape(hd))
    ab_row = jnp.zeros((1, 128), jnp.float32).at[0, :h].set(aA_b[:, 0])
    bf = jnp.bfloat16
    wcat = wcat.astype(bf)
    abd = abd.astype(bf)
    n2_wb = n2_w.astype(bf)
    n3_wb = n3_w.astype(bf)
    d1_wb = d1_w.astype(bf)
    d2_wb = d2_w.astype(bf)
    e1_wb = e1_w.astype(bf)
    e2_wb = e2_w.astype(bf)
    e3_wb = e3_w.astype(bf)

    gath2 = _sc_gather_rows(bb_nodes, idx_both3)

    ew, nu = _edge_compute(
        gath2, eattr, wcat, bcat, abd, ab_row, n2_wb, n2_b.reshape(1, d), n3_wb,
        n3_b.reshape(1, d), e_num, be_blk)

    normtab = _sc_scatter_add(ew, snk3, n_num)

    rnorm = _combine_norm(normtab, n_num, bn_blk)

    rn = _sc_gather_rows(rnorm, snk3)

    wnu = _apply_atten(ew, rn, nu, e_num, be_blk)

    aggflat = _sc_scatter_add(wnu, snk3, n_num)

    bb_out = _node_update(
        aggflat, bb_nodes, d1_wb, d1_b.reshape(1, 4 * d), d2_wb,
        d2_b.reshape(1, d), g1.reshape(1, d), b1.reshape(1, d), n_num, bn_blk)

    gath2b = _sc_gather_rows(bb_out, idx_both3)

    eattr_out = _edge_update(
        gath2b, eattr, e1_wb, e1_b.reshape(1, de), e2_wb, e2_b.reshape(1, de),
        e3_wb, e3_b.reshape(1, de), ge.reshape(1, de), be.reshape(1, de),
        e_num, be_blk)

    return (bb_out, eattr_out)


# speedup vs baseline: 1.0217x; 1.0217x over previous
import functools

import jax
import jax.numpy as jnp
from jax import lax
from jax.experimental import pallas as pl
from jax.experimental.pallas import tpu as pltpu
from jax.experimental.pallas import tpu_sc as plsc

_NC = 2
_NS = 16
_NW = _NC * _NS
_IW = 128


def _gelu(x):
    return 0.5 * x * (1.0 + lax.erf(x * 0.7071067811865476))


def _ln(x, g, b):
    m = jnp.mean(x, axis=-1, keepdims=True)
    v = jnp.mean((x - m) ** 2, axis=-1, keepdims=True)
    return (x - m) / jnp.sqrt(v + 1e-5) * g + b


def _sc_gather_rows(table, idx3):
    _, dt = table.shape
    nq, gsub, iw = idx3.shape
    m = nq * gsub * iw
    ge = gsub * iw
    mesh = plsc.VectorSubcoreMesh(core_axis_name="c", subcore_axis_name="s")

    @functools.partial(
        pl.kernel, mesh=mesh,
        out_type=jax.ShapeDtypeStruct((m, dt), table.dtype),
        scratch_types=[
            pltpu.VMEM((gsub, iw), jnp.int32),
            pltpu.VMEM((ge, dt), table.dtype),
            pltpu.SemaphoreType.DMA,
        ],
    )
    def k(table_hbm, idx_hbm, out_hbm, idx_v, rows_v, sem):
        wid = lax.axis_index("s") * _NC + lax.axis_index("c")

        @pl.loop(wid, nq, step=_NW)
        def _(q):
            pltpu.sync_copy(idx_hbm.at[q], idx_v)
            waits = [
                pltpu.async_copy(table_hbm.at[idx_v.at[j]],
                                 rows_v.at[pl.ds(j * iw, iw)], sem)
                for j in range(gsub)
            ]
            for w in waits:
                w.wait()
            pltpu.sync_copy(rows_v, out_hbm.at[pl.ds(q * ge, ge)])

    return k(table, idx3)


_GSUB = 4
_GE = _GSUB * _IW
_ZC = 40


def _sc_scatter_add(vals, idx3, nseg):
    m, dv = vals.shape
    ngrp, gsub, iw = idx3.shape
    ge = gsub * iw
    assert dv == 128 and m == ngrp * ge
    half = nseg // _NC
    acc_rows = half + _ZC
    nzc = acc_rows // _ZC
    nxc = half // _ZC
    assert half % _ZC == 0
    mesh = plsc.VectorSubcoreMesh(core_axis_name="c", subcore_axis_name="s")

    @functools.partial(
        pl.kernel, mesh=mesh,
        out_type=jax.ShapeDtypeStruct((nseg, dv), jnp.float32),
        scratch_types=[
            pltpu.VMEM((2, gsub, iw), jnp.int32),
            pltpu.VMEM((2, ge, dv), jnp.float32),
            pltpu.VMEM((_ZC, dv), jnp.float32),
            pltpu.VMEM_SHARED((acc_rows, dv), jnp.float32),
            pltpu.SemaphoreType.DMA,
            pltpu.SemaphoreType.DMA,
        ],
    )
    def k(vals_hbm, idx_hbm, out_hbm, idx_v, data_v, zero_v, acc_sh, sem_a, sem_b):
        cid = lax.axis_index("c")
        sid = lax.axis_index("s")
        lo = cid * half

        @pl.loop(0, _ZC)
        def _(r):
            @pl.loop(0, dv, step=16)
            def _(c0):
                zero_v[r, pl.ds(c0, 16)] = jnp.zeros((16,), jnp.float32)

        @pl.loop(sid, nzc, step=_NS)
        def _(c0):
            pltpu.sync_copy(zero_v, acc_sh.at[pl.ds(c0 * _ZC, _ZC)])

        plsc.subcore_barrier()

        def remap_and_scatter(b, g):
            @pl.loop(0, gsub)
            def _(j):
                @pl.loop(0, iw, step=16)
                def _(k0):
                    v = idx_v[b, j, pl.ds(k0, 16)] - lo
                    bad = (v < 0) | (v >= half)
                    idx_v[b, j, pl.ds(k0, 16)] = jnp.where(bad, half, v)

            for j in range(gsub):
                pltpu.sync_copy(data_v.at[b, pl.ds(j * iw, iw)],
                                acc_sh.at[idx_v.at[b, j]], add=True)

        @pl.loop(sid, ngrp, step=2 * _NS)
        def _(g0):
            g1 = g0 + _NS
            g1c = jnp.minimum(g1, ngrp - 1)
            w0i = pltpu.async_copy(idx_hbm.at[g0], idx_v.at[0], sem_a)
            w0d = pltpu.async_copy(vals_hbm.at[pl.ds(g0 * ge, ge)],
                                   data_v.at[0], sem_a)
            w1i = pltpu.async_copy(idx_hbm.at[g1c], idx_v.at[1], sem_b)
            w1d = pltpu.async_copy(vals_hbm.at[pl.ds(g1c * ge, ge)],
                                   data_v.at[1], sem_b)
            w0i.wait()
            w0d.wait()
            remap_and_scatter(0, g0)
            w1i.wait()
            w1d.wait()

            @pl.when(g1 < ngrp)
            def _():
                remap_and_scatter(1, g1)

        plsc.subcore_barrier()

        @pl.loop(sid, nxc, step=_NS)
        def _(c0):
            pltpu.sync_copy(
                acc_sh.at[pl.ds(c0 * _ZC, _ZC)],
                out_hbm.at[pl.ds(lo + c0 * _ZC, _ZC)],
            )

    return k(vals, idx3)


def _edge_compute(gath2, eattr, wcat, bcat, abd, ab_row, n2_w, nb2, n3_w, nb3,
                  e_num, be_blk):
    nblk = e_num // be_blk
    d = 128
    hd = 512

    bf = jnp.bfloat16

    def body(gs_r, gk_r, ea_r, w_r, b_r, a_r, ab_r, n2_r, nb2_r, n3_r, nb3_r,
             ew_r, nu_r):
        pre = (
            jnp.dot(gs_r[...].astype(bf), w_r[0:d, :], preferred_element_type=jnp.float32)
            + jnp.dot(gk_r[...].astype(bf), w_r[d:2 * d, :], preferred_element_type=jnp.float32)
            + jnp.dot(ea_r[...].astype(bf), w_r[2 * d:, :], preferred_element_type=jnp.float32)
            + b_r[...]
        )
        att = pre[:, 0:hd]
        act = jnp.where(att >= 0, att, 0.2 * att)
        aw = jnp.dot(act.astype(bf), a_r[...], preferred_element_type=jnp.float32) + ab_r[...]
        lane = lax.broadcasted_iota(jnp.int32, (be_blk, 128), 1)
        ew_r[...] = jnp.where(lane < 4, jnp.exp(aw) + 1e-12, 0.0)
        h = _gelu(pre[:, hd:hd + d])
        h = _gelu(jnp.dot(h.astype(bf), n2_r[...], preferred_element_type=jnp.float32) + nb2_r[...])
        nu_r[...] = (jnp.dot(h.astype(bf), n3_r[...], preferred_element_type=jnp.float32) + nb3_r[...]).astype(bf)

    return pl.pallas_call(
        body,
        grid=(nblk,),
        in_specs=[
            pl.BlockSpec((be_blk, d), lambda i: (i, 0)),
            pl.BlockSpec((be_blk, d), lambda i, n=nblk: (i + n, 0)),
            pl.BlockSpec((be_blk, 16), lambda i: (i, 0)),
            pl.BlockSpec(wcat.shape, lambda i: (0, 0)),
            pl.BlockSpec(bcat.shape, lambda i: (0, 0)),
            pl.BlockSpec(abd.shape, lambda i: (0, 0)),
            pl.BlockSpec(ab_row.shape, lambda i: (0, 0)),
            pl.BlockSpec(n2_w.shape, lambda i: (0, 0)),
            pl.BlockSpec(nb2.shape, lambda i: (0, 0)),
            pl.BlockSpec(n3_w.shape, lambda i: (0, 0)),
            pl.BlockSpec(nb3.shape, lambda i: (0, 0)),
        ],
        out_specs=[
            pl.BlockSpec((be_blk, 128), lambda i: (i, 0)),
            pl.BlockSpec((be_blk, d), lambda i: (i, 0)),
        ],
        out_shape=[
            jax.ShapeDtypeStruct((e_num, 128), jnp.float32),
            jax.ShapeDtypeStruct((e_num, 128), jnp.bfloat16),
        ],
    )(gath2, gath2, eattr, wcat, bcat, abd, ab_row, n2_w, nb2, n3_w, nb3)


def _combine_norm(normtab, n_num, bn_blk):
    nblk = n_num // bn_blk

    def body(a_r, o_r):
        s = a_r[...]
        o_r[...] = jnp.where(s > 0, 0.25 / s, 0.0)

    return pl.pallas_call(
        body,
        grid=(nblk,),
        in_specs=[pl.BlockSpec((bn_blk, 128), lambda i: (i, 0))],
        out_specs=pl.BlockSpec((bn_blk, 128), lambda i: (i, 0)),
        out_shape=jax.ShapeDtypeStruct((n_num, 128), jnp.float32),
    )(normtab)


def _apply_atten(ew, rn, nu, e_num, be_blk):
    nblk = e_num // be_blk

    def body(ew_r, rn_r, nu_r, o_r):
        w = jnp.sum(ew_r[...] * rn_r[...], axis=1, keepdims=True)
        o_r[...] = w * nu_r[...].astype(jnp.float32)

    return pl.pallas_call(
        body,
        grid=(nblk,),
        in_specs=[
            pl.BlockSpec((be_blk, 128), lambda i: (i, 0)),
            pl.BlockSpec((be_blk, 128), lambda i: (i, 0)),
            pl.BlockSpec((be_blk, 128), lambda i: (i, 0)),
        ],
        out_specs=pl.BlockSpec((be_blk, 128), lambda i: (i, 0)),
        out_shape=jax.ShapeDtypeStruct((e_num, 128), jnp.float32),
    )(ew, rn, nu)


def _node_update(aggflat, bb, d1_w, d1_b, d2_w, d2_b, g1, b1, n_num, bn_blk):
    nblk = n_num // bn_blk

    bf = jnp.bfloat16

    def body(a0_r, bb_r, d1_r, d1b_r, d2_r, d2b_r, g_r, bb1_r, o_r):
        upd = a0_r[...]
        x = _ln(bb_r[...] + upd, g_r[...], bb1_r[...])
        h = _gelu(jnp.dot(x.astype(bf), d1_r[...], preferred_element_type=jnp.float32) + d1b_r[...])
        dense = jnp.dot(h.astype(bf), d2_r[...], preferred_element_type=jnp.float32) + d2b_r[...]
        o_r[...] = _ln(dense + upd, g_r[...], bb1_r[...])

    return pl.pallas_call(
        body,
        grid=(nblk,),
        in_specs=[
            pl.BlockSpec((bn_blk, 128), lambda i: (i, 0)),
            pl.BlockSpec((bn_blk, 128), lambda i: (i, 0)),
            pl.BlockSpec(d1_w.shape, lambda i: (0, 0)),
            pl.BlockSpec(d1_b.shape, lambda i: (0, 0)),
            pl.BlockSpec(d2_w.shape, lambda i: (0, 0)),
            pl.BlockSpec(d2_b.shape, lambda i: (0, 0)),
            pl.BlockSpec(g1.shape, lambda i: (0, 0)),
            pl.BlockSpec(b1.shape, lambda i: (0, 0)),
        ],
        out_specs=pl.BlockSpec((bn_blk, 128), lambda i: (i, 0)),
        out_shape=jax.ShapeDtypeStruct((n_num, 128), jnp.float32),
    )(aggflat, bb, d1_w, d1_b, d2_w, d2_b, g1, b1)


def _edge_update(gath2, eattr, e1_w, e1_b, e2_w, e2_b, e3_w, e3_b, ge, be, e_num, be_blk):
    nblk = e_num // be_blk
    d = 128

    bf = jnp.bfloat16

    def body(gs_r, gk_r, ea_r, e1_r, e1b_r, e2_r, e2b_r, e3_r, e3b_r, g_r, b_r, o_r):
        t = (
            jnp.dot(gs_r[...].astype(bf), e1_r[0:d, :], preferred_element_type=jnp.float32)
            + jnp.dot(gk_r[...].astype(bf), e1_r[d:2 * d, :], preferred_element_type=jnp.float32)
            + jnp.dot(ea_r[...].astype(bf), e1_r[2 * d:, :], preferred_element_type=jnp.float32)
            + e1b_r[...]
        )
        t = _gelu(t)
        t = _gelu(jnp.dot(t.astype(bf), e2_r[...], preferred_element_type=jnp.float32) + e2b_r[...])
        t = jnp.dot(t.astype(bf), e3_r[...], preferred_element_type=jnp.float32) + e3b_r[...]
        o_r[...] = _ln(ea_r[...] + t, g_r[...], b_r[...])

    return pl.pallas_call(
        body,
        grid=(nblk,),
        in_specs=[
            pl.BlockSpec((be_blk, d), lambda i: (i, 0)),
            pl.BlockSpec((be_blk, d), lambda i, n=nblk: (i + n, 0)),
            pl.BlockSpec((be_blk, 16), lambda i: (i, 0)),
            pl.BlockSpec(e1_w.shape, lambda i: (0, 0)),
            pl.BlockSpec(e1_b.shape, lambda i: (0, 0)),
            pl.BlockSpec(e2_w.shape, lambda i: (0, 0)),
            pl.BlockSpec(e2_b.shape, lambda i: (0, 0)),
            pl.BlockSpec(e3_w.shape, lambda i: (0, 0)),
            pl.BlockSpec(e3_b.shape, lambda i: (0, 0)),
            pl.BlockSpec(ge.shape, lambda i: (0, 0)),
            pl.BlockSpec(be.shape, lambda i: (0, 0)),
        ],
        out_specs=pl.BlockSpec((be_blk, 16), lambda i: (i, 0)),
        out_shape=jax.ShapeDtypeStruct((e_num, 16), jnp.float32),
    )(gath2, gath2, eattr, e1_w, e1_b, e2_w, e2_b, e3_w, e3_b, ge, be)


def kernel(bb_nodes, eidx, eattr, aW_w, aW_b, aA_w, aA_b, n1_w, n1_b, n2_w, n2_b,
           n3_w, n3_b, d1_w, d1_b, d2_w, d2_b, e1_w, e1_b, e2_w, e2_b, e3_w, e3_b,
           g1, b1, ge, be):
    n_num, d = bb_nodes.shape
    e_num, de = eattr.shape
    h = aW_w.shape[0]
    hd = h * d
    be_blk = 2560
    bn_blk = 1000

    snk = eidx[1]
    snk3 = snk.reshape(e_num // 256, 2, _IW)
    idx_both3 = eidx.reshape(2 * e_num // _GE, _GSUB, _IW)

    wcat = jnp.concatenate(
        [jnp.transpose(aW_w, (1, 0, 2)).reshape(2 * d + de, hd), n1_w], axis=1)
    bcat = jnp.concatenate(
        [aW_b.reshape(1, hd), n1_b.reshape(1, d)], axis=1)
    abd = jnp.zeros((hd, 128), jnp.float32).at[
        jnp.arange(hd), jnp.repeat(jnp.arange(h), d)].set(aA_w.reshape(hd))
    ab_row = jnp.zeros((1, 128), jnp.float32).at[0, :h].set(aA_b[:, 0])
    bf = jnp.bfloat16
    wcat = wcat.astype(bf)
    abd = abd.astype(bf)
    n2_wb = n2_w.astype(bf)
    n3_wb = n3_w.astype(bf)
    d1_wb = d1_w.astype(bf)
    d2_wb = d2_w.astype(bf)
    e1_wb = e1_w.astype(bf)
    e2_wb = e2_w.astype(bf)
    e3_wb = e3_w.astype(bf)

    gath2 = _sc_gather_rows(bb_nodes, idx_both3)

    ew, nu = _edge_compute(
        gath2, eattr, wcat, bcat, abd, ab_row, n2_wb, n2_b.reshape(1, d), n3_wb,
        n3_b.reshape(1, d), e_num, be_blk)

    normtab = _sc_scatter_add(ew, snk3, n_num)

    rnorm = _combine_norm(normtab, n_num, bn_blk)

    rn = _sc_gather_rows(rnorm, snk3)

    wnu = _apply_atten(ew, rn, nu, e_num, be_blk)

    aggflat = _sc_scatter_add(wnu, snk3, n_num)

    bb_out = _node_update(
        aggflat, bb_nodes, d1_wb, d1_b.reshape(1, 4 * d), d2_wb,
        d2_b.reshape(1, d), g1.reshape(1, d), b1.reshape(1, d), n_num, bn_blk)

    gath2b = _sc_gather_rows(bb_out, idx_both3)

    eattr_out = _edge_update(
        gath2b, eattr, e1_wb, e1_b.reshape(1, de), e2_wb, e2_b.reshape(1, de),
        e3_wb, e3_b.reshape(1, de), ge.reshape(1, de), be.reshape(1, de),
        e_num, be_blk)

    return (bb_out, eattr_out)
